# baseline (speedup 1.0000x reference)
import functools

import jax
import jax.numpy as jnp
import numpy as np
from jax import lax
from jax.experimental import pallas as pl
from jax.experimental.pallas import tpu as pltpu
from jax.experimental.pallas import tpu_sc as plsc

_NUM_PLAYERS = 100000
_BATCH = 16384
_S = float(np.log(10.0) / 800.0)

_NC = 2
_NS = 16
_L = 16
_NW = _NC * _NS
_MPW = _BATCH // _NW
_CH = 128
_NCH = _MPW // _CH


@functools.partial(
    pl.kernel,
    out_type=(
        jax.ShapeDtypeStruct((_BATCH,), jnp.float32),
        jax.ShapeDtypeStruct((_BATCH,), jnp.float32),
        jax.ShapeDtypeStruct((_BATCH,), jnp.float32),
    ),
    mesh=plsc.VectorSubcoreMesh(core_axis_name="c", subcore_axis_name="s"),
    compiler_params=pltpu.CompilerParams(needs_layout_passes=False),
    scratch_types=[
        pltpu.VMEM((_NCH, _CH), jnp.int32),
        pltpu.VMEM((_NCH, _CH), jnp.int32),
        pltpu.VMEM((_MPW,), jnp.float32),
        pltpu.VMEM((_MPW,), jnp.float32),
        pltpu.VMEM((1,), jnp.float32),
        pltpu.VMEM((1,), jnp.float32),
        pltpu.VMEM((_MPW,), jnp.float32),
        pltpu.VMEM((_MPW,), jnp.float32),
        pltpu.VMEM((_MPW,), jnp.float32),
        pltpu.SemaphoreType.DMA,
        pltpu.SemaphoreType.DMA,
        pltpu.SemaphoreType.DMA,
        pltpu.SemaphoreType.DMA,
        pltpu.SemaphoreType.DMA,
    ],
)
def _elo_sc(x_hbm, ratings_hbm, k_hbm, b_hbm, p1_hbm, dr_hbm, p2_hbm,
            idx1_v, idx2_v, g1_v, g2_v, k_sm, b_sm, p1_v, dr_v, p2_v,
            sem_i1, sem_i2, sem_kb, sem_g, sem_o):
    wid = lax.axis_index("s") * _NC + lax.axis_index("c")
    base = wid * _MPW

    ci1 = [pltpu.async_copy(
        x_hbm.at[0].at[pl.ds(base + j * _CH, _CH)], idx1_v.at[j], sem_i1)
        for j in range(_NCH)]
    ci2 = [pltpu.async_copy(
        x_hbm.at[1].at[pl.ds(base + j * _CH, _CH)], idx2_v.at[j], sem_i2)
        for j in range(_NCH)]
    ck = pltpu.async_copy(k_hbm, k_sm, sem_kb)
    cb = pltpu.async_copy(b_hbm, b_sm, sem_kb)

    gathers = []
    for c in ci1:
        c.wait()
    for j in range(_NCH):
        gathers.append(pltpu.async_copy(
            ratings_hbm.at[idx1_v.at[j]], g1_v.at[pl.ds(j * _CH, _CH)], sem_g))
    for c in ci2:
        c.wait()
    for j in range(_NCH):
        gathers.append(pltpu.async_copy(
            ratings_hbm.at[idx2_v.at[j]], g2_v.at[pl.ds(j * _CH, _CH)], sem_g))

    ck.wait()
    cb.wait()
    zeros = jnp.zeros((_L,), jnp.int32)
    kvec = plsc.load_gather(k_sm, [zeros])
    bvec = plsc.load_gather(b_sm, [zeros])

    def fill(i, carry):
        dr_v[pl.ds(i * _L, _L)] = kvec
        return carry

    lax.fori_loop(0, _MPW // _L, fill, 0)
    cdr = pltpu.async_copy(dr_v, dr_hbm.at[pl.ds(base, _MPW)], sem_o)

    for c in gathers:
        c.wait()

    def body(i, carry):
        o = i * _L
        r1 = g1_v[pl.ds(o, _L)]
        r2 = g2_v[pl.ds(o, _L)]
        p1 = _S * (r1 - r2) + bvec
        p1_v[pl.ds(o, _L)] = p1
        p2_v[pl.ds(o, _L)] = -p1
        return carry

    lax.fori_loop(0, _MPW // _L, body, 0)

    cp1 = pltpu.async_copy(p1_v, p1_hbm.at[pl.ds(base, _MPW)], sem_o)
    cp2 = pltpu.async_copy(p2_v, p2_hbm.at[pl.ds(base, _MPW)], sem_o)
    cdr.wait()
    cp1.wait()
    cp2.wait()


def kernel(x, ratings, k, b):
    xr = x.astype(jnp.int32)
    k1 = k.astype(jnp.float32).reshape(1)
    b1 = b.astype(jnp.float32).reshape(1)
    p1, dr, p2 = _elo_sc(xr, ratings, k1, b1)
    return jnp.stack([p1, dr, p2], axis=1)


# speedup vs baseline: 1.0233x; 1.0233x over previous
import functools

import jax
import jax.numpy as jnp
import numpy as np
from jax import lax
from jax.experimental import pallas as pl
from jax.experimental.pallas import tpu as pltpu
from jax.experimental.pallas import tpu_sc as plsc

_NUM_PLAYERS = 100000
_BATCH = 16384
_S = float(np.log(10.0) / 800.0)

_NC = 2
_NS = 16
_L = 16
_NW = _NC * _NS
_MPW = _BATCH // _NW
_CH = 128
_NCH = _MPW // _CH


@functools.partial(
    pl.kernel,
    out_type=(
        jax.ShapeDtypeStruct((_BATCH,), jnp.float32),
        jax.ShapeDtypeStruct((_BATCH,), jnp.float32),
    ),
    mesh=plsc.VectorSubcoreMesh(core_axis_name="c", subcore_axis_name="s"),
    compiler_params=pltpu.CompilerParams(needs_layout_passes=False),
    scratch_types=[
        pltpu.VMEM((_NCH, _CH), jnp.int32),
        pltpu.VMEM((_NCH, _CH), jnp.int32),
        pltpu.VMEM((_MPW,), jnp.float32),
        pltpu.VMEM((_MPW,), jnp.float32),
        pltpu.VMEM((1,), jnp.float32),
        pltpu.VMEM((_MPW,), jnp.float32),
        pltpu.VMEM((_MPW,), jnp.float32),
        pltpu.SemaphoreType.DMA,
        pltpu.SemaphoreType.DMA,
        pltpu.SemaphoreType.DMA,
        pltpu.SemaphoreType.DMA,
        pltpu.SemaphoreType.DMA,
    ],
)
def _elo_sc(x_hbm, ratings_hbm, b_hbm, p1_hbm, p2_hbm,
            idx1_v, idx2_v, g1_v, g2_v, b_sm, p1_v, p2_v,
            sem_i1, sem_i2, sem_kb, sem_g, sem_o):
    wid = lax.axis_index("s") * _NC + lax.axis_index("c")
    base = wid * _MPW

    ci1 = [pltpu.async_copy(x_hbm.at[wid], idx1_v, sem_i1)]
    ci2 = [pltpu.async_copy(x_hbm.at[_NW + wid], idx2_v, sem_i2)]
    cb = pltpu.async_copy(b_hbm, b_sm, sem_kb)

    gathers = []
    for c in ci1:
        c.wait()
    for j in range(_NCH):
        gathers.append(pltpu.async_copy(
            ratings_hbm.at[idx1_v.at[j]], g1_v.at[pl.ds(j * _CH, _CH)], sem_g))
    for c in ci2:
        c.wait()
    for j in range(_NCH):
        gathers.append(pltpu.async_copy(
            ratings_hbm.at[idx2_v.at[j]], g2_v.at[pl.ds(j * _CH, _CH)], sem_g))

    cb.wait()
    zeros = jnp.zeros((_L,), jnp.int32)
    bvec = plsc.load_gather(b_sm, [zeros])

    for c in gathers:
        c.wait()

    def body(i, carry):
        o = i * _L
        r1 = g1_v[pl.ds(o, _L)]
        r2 = g2_v[pl.ds(o, _L)]
        p1 = _S * (r1 - r2) + bvec
        p1_v[pl.ds(o, _L)] = p1
        p2_v[pl.ds(o, _L)] = -p1
        return carry

    lax.fori_loop(0, _MPW // _L, body, 0)

    cp1 = pltpu.async_copy(p1_v, p1_hbm.at[pl.ds(base, _MPW)], sem_o)
    cp2 = pltpu.async_copy(p2_v, p2_hbm.at[pl.ds(base, _MPW)], sem_o)
    cp1.wait()
    cp2.wait()


def kernel(x, ratings, k, b):
    xr = x.astype(jnp.int32).reshape(2 * _NW, _NCH, _CH)
    b1 = b.astype(jnp.float32).reshape(1)
    p1, p2 = _elo_sc(xr, ratings, b1)
    dr = jnp.zeros_like(p1) + k.astype(jnp.float32)
    return jnp.stack([p1, dr, p2], axis=1)


# speedup vs baseline: 1.0267x; 1.0034x over previous
import functools

import jax
import jax.numpy as jnp
import numpy as np
from jax import lax
from jax.experimental import pallas as pl
from jax.experimental.pallas import tpu as pltpu
from jax.experimental.pallas import tpu_sc as plsc

_NUM_PLAYERS = 100000
_BATCH = 16384
_S = float(np.log(10.0) / 800.0)

_NC = 2
_NS = 16
_L = 16
_NW = _NC * _NS
_MPW = _BATCH // _NW
_CH = 128
_NCH = _MPW // _CH


@functools.partial(
    pl.kernel,
    out_type=(
        jax.ShapeDtypeStruct((_BATCH,), jnp.float32),
        jax.ShapeDtypeStruct((_BATCH,), jnp.float32),
    ),
    mesh=plsc.VectorSubcoreMesh(core_axis_name="c", subcore_axis_name="s"),
    compiler_params=pltpu.CompilerParams(needs_layout_passes=False),
    scratch_types=[
        pltpu.VMEM((_NCH, _CH), jnp.int32),
        pltpu.VMEM((_NCH, _CH), jnp.int32),
        pltpu.VMEM((_MPW,), jnp.float32),
        pltpu.VMEM((_MPW,), jnp.float32),
        pltpu.VMEM((1,), jnp.float32),
        pltpu.VMEM((_MPW,), jnp.float32),
        pltpu.VMEM((_MPW,), jnp.float32),
        pltpu.SemaphoreType.DMA,
        pltpu.SemaphoreType.DMA,
        pltpu.SemaphoreType.DMA,
        pltpu.SemaphoreType.DMA,
        pltpu.SemaphoreType.DMA,
    ],
)
def _elo_sc(x_hbm, ratings_hbm, b_hbm, p1_hbm, p2_hbm,
            idx1_v, idx2_v, g1_v, g2_v, b_sm, p1_v, p2_v,
            sem_i1, sem_i2, sem_kb, sem_g, sem_o):
    wid = lax.axis_index("s") * _NC + lax.axis_index("c")
    base = wid * _MPW

    ci1 = [pltpu.async_copy(x_hbm.at[pl.ds(_NCH * wid, _NCH)], idx1_v, sem_i1)]
    ci2 = [pltpu.async_copy(
        x_hbm.at[pl.ds(_BATCH // _CH + _NCH * wid, _NCH)], idx2_v, sem_i2)]
    cb = pltpu.async_copy(b_hbm, b_sm, sem_kb)

    gathers = []
    for c in ci1:
        c.wait()
    for j in range(_NCH):
        gathers.append(pltpu.async_copy(
            ratings_hbm.at[idx1_v.at[j]], g1_v.at[pl.ds(j * _CH, _CH)], sem_g))
    for c in ci2:
        c.wait()
    for j in range(_NCH):
        gathers.append(pltpu.async_copy(
            ratings_hbm.at[idx2_v.at[j]], g2_v.at[pl.ds(j * _CH, _CH)], sem_g))

    cb.wait()
    zeros = jnp.zeros((_L,), jnp.int32)
    bvec = plsc.load_gather(b_sm, [zeros])

    for c in gathers:
        c.wait()

    def body(i, carry):
        o = i * _L
        r1 = g1_v[pl.ds(o, _L)]
        r2 = g2_v[pl.ds(o, _L)]
        p1 = _S * (r1 - r2) + bvec
        p1_v[pl.ds(o, _L)] = p1
        p2_v[pl.ds(o, _L)] = -p1
        return carry

    lax.fori_loop(0, _MPW // _L, body, 0)

    cp1 = pltpu.async_copy(p1_v, p1_hbm.at[pl.ds(base, _MPW)], sem_o)
    cp2 = pltpu.async_copy(p2_v, p2_hbm.at[pl.ds(base, _MPW)], sem_o)
    cp1.wait()
    cp2.wait()


def kernel(x, ratings, k, b):
    xr = x.astype(jnp.int32).reshape(2 * _BATCH // _CH, _CH)
    b1 = b.astype(jnp.float32).reshape(1)
    p1, p2 = _elo_sc(xr, ratings, b1)
    dr = jnp.zeros_like(p1) + k.astype(jnp.float32)
    return jnp.stack([p1, dr, p2], axis=1)


# speedup vs baseline: 1.0402x; 1.0131x over previous
import functools

import jax
import jax.numpy as jnp
import numpy as np
from jax import lax
from jax.experimental import pallas as pl
from jax.experimental.pallas import tpu as pltpu
from jax.experimental.pallas import tpu_sc as plsc

_NUM_PLAYERS = 100000
_BATCH = 16384
_S = float(np.log(10.0) / 800.0)

_NC = 2
_NS = 16
_L = 16
_NW = _NC * _NS
_MPW = _BATCH // _NW
_CH = 128
_NCH = _MPW // _CH


@functools.partial(
    pl.kernel,
    out_type=(
        jax.ShapeDtypeStruct((_BATCH,), jnp.float32),
        jax.ShapeDtypeStruct((_BATCH,), jnp.float32),
    ),
    mesh=plsc.VectorSubcoreMesh(core_axis_name="c", subcore_axis_name="s"),
    compiler_params=pltpu.CompilerParams(needs_layout_passes=False),
    scratch_types=[
        pltpu.VMEM((_NCH, _CH), jnp.int32),
        pltpu.VMEM((_NCH, _CH), jnp.int32),
        pltpu.VMEM((_MPW,), jnp.float32),
        pltpu.VMEM((_MPW,), jnp.float32),
        pltpu.VMEM((1,), jnp.float32),
        pltpu.VMEM((_MPW,), jnp.float32),
        pltpu.VMEM((_MPW,), jnp.float32),
        pltpu.SemaphoreType.DMA,
        pltpu.SemaphoreType.DMA,
        pltpu.SemaphoreType.DMA,
        pltpu.SemaphoreType.DMA,
        pltpu.SemaphoreType.DMA,
        pltpu.SemaphoreType.DMA,
        pltpu.SemaphoreType.DMA,
        pltpu.SemaphoreType.DMA,
    ],
)
def _elo_sc(x_hbm, ratings_hbm, b_hbm, p1_hbm, p2_hbm,
            idx1_v, idx2_v, g1_v, g2_v, b_sm, p1_v, p2_v,
            sem_i1, sem_i2, sem_kb, sem_c0, sem_c1, sem_c2, sem_c3, sem_o):
    wid = lax.axis_index("s") * _NC + lax.axis_index("c")
    base = wid * _MPW
    sem_c = [sem_c0, sem_c1, sem_c2, sem_c3]

    ci1 = pltpu.async_copy(x_hbm.at[pl.ds(_NCH * wid, _NCH)], idx1_v, sem_i1)
    ci2 = pltpu.async_copy(
        x_hbm.at[pl.ds(_BATCH // _CH + _NCH * wid, _NCH)], idx2_v, sem_i2)
    cb = pltpu.async_copy(b_hbm, b_sm, sem_kb)

    g1c, g2c = [], []
    ci1.wait()
    for j in range(_NCH):
        g1c.append(pltpu.async_copy(
            ratings_hbm.at[idx1_v.at[j]], g1_v.at[pl.ds(j * _CH, _CH)], sem_c[j]))
    ci2.wait()
    for j in range(_NCH):
        g2c.append(pltpu.async_copy(
            ratings_hbm.at[idx2_v.at[j]], g2_v.at[pl.ds(j * _CH, _CH)], sem_c[j]))

    cb.wait()
    zeros = jnp.zeros((_L,), jnp.int32)
    bvec = plsc.load_gather(b_sm, [zeros])

    outs = []
    for j in range(_NCH):
        g1c[j].wait()
        g2c[j].wait()
        for t in range(_CH // _L):
            o = j * _CH + t * _L
            r1 = g1_v[pl.ds(o, _L)]
            r2 = g2_v[pl.ds(o, _L)]
            p1 = _S * (r1 - r2) + bvec
            p1_v[pl.ds(o, _L)] = p1
            p2_v[pl.ds(o, _L)] = -p1
        outs.append(pltpu.async_copy(
            p1_v.at[pl.ds(j * _CH, _CH)],
            p1_hbm.at[pl.ds(base + j * _CH, _CH)], sem_o))
        outs.append(pltpu.async_copy(
            p2_v.at[pl.ds(j * _CH, _CH)],
            p2_hbm.at[pl.ds(base + j * _CH, _CH)], sem_o))

    for c in outs:
        c.wait()


def kernel(x, ratings, k, b):
    xr = x.astype(jnp.int32).reshape(2 * _BATCH // _CH, _CH)
    b1 = b.astype(jnp.float32).reshape(1)
    p1, p2 = _elo_sc(xr, ratings, b1)
    dr = jnp.zeros_like(p1) + k.astype(jnp.float32)
    return jnp.stack([p1, dr, p2], axis=1)
